# baseline (speedup 1.0000x reference)
import numpy as np
import jax
import jax.numpy as jnp
from jax.experimental import pallas as pl

H = 32
DH = 128
D = 4096
DFF = 11008
KV_LEN = 4095
S = KV_LEN + 1
KB = 2048
NUM_REMAIN = max(min(S, 128), S - int(S * 0.9))
_EPS = 1e-5


def _rope_tables_np():
    inv_freq = 1.0 / (10000.0 ** (np.arange(0, DH, 2, dtype=np.float64) / DH))
    pos = np.arange(S, dtype=np.float64)
    freqs = np.outer(pos, inv_freq)
    emb = np.concatenate([freqs, freqs], axis=-1)
    sin = np.sin(emb)
    sin[:, : DH // 2] *= -1.0
    return np.cos(emb).astype(np.float32), sin.astype(np.float32)


def _roll(x):
    x1 = x[..., : DH // 2]
    x2 = x[..., DH // 2:]
    return jnp.concatenate([x2, x1], axis=-1)


def _dot_t(a, b):
    return jax.lax.dot_general(a, b, (((1,), (1,)), ((), ())),
                               preferred_element_type=jnp.float32)


QB = 2


def _qkv_kernel(hid_ref, ln1_ref, wq_ref, wk_ref, wv_ref, cos_ref, sin_ref,
                hp1_ref, hb1_ref, hp2_ref, hb2_ref,
                q_ref, k_ref, v_ref, qh_ref):
    h = hid_ref[...]
    s = jax.lax.rsqrt(jnp.mean(h * h) + _EPS)
    hn = h * s * ln1_ref[...]
    q = jnp.dot(hn, wq_ref[...], preferred_element_type=jnp.float32)
    k = jnp.dot(hn, wk_ref[...], preferred_element_type=jnp.float32)
    v = jnp.dot(hn, wv_ref[...], preferred_element_type=jnp.float32)
    c = cos_ref[...]
    sn = sin_ref[...]
    for i in range(QB):
        sl = slice(i * DH, (i + 1) * DH)
        qi, ki, vi = q[:, sl], k[:, sl], v[:, sl]
        qr = qi * c + _roll(qi) * sn
        q_ref[i] = qr
        k_ref[i] = ki * c + _roll(ki) * sn
        v_ref[i] = vi
        hp1 = hp1_ref[0, i]
        hp2 = hp2_ref[0, i]
        dq = jax.nn.silu(jnp.dot(qr, hp1, preferred_element_type=jnp.float32)
                         + hb1_ref[0, i])
        q1 = dq + qr
        qh_ref[i] = jnp.sign(jnp.dot(q1, hp2, preferred_element_type=jnp.float32)
                             + hb2_ref[0, i] + q1)


def _key_kernel(kc_ref, cos_ref, sin_ref, hp1_ref, hb1_ref, hp2_ref, hb2_ref,
                qr_ref, knew_ref, qh_ref, draft_ref, score_ref):
    b = pl.program_id(1)
    kb = kc_ref[0, 0]
    c = cos_ref[...]
    sn = sin_ref[...]
    kr = kb * c + _roll(kb) * sn
    rows = b * KB + jax.lax.broadcasted_iota(jnp.int32, (KB, 1), 0)
    kr = jnp.where(rows == S - 1, knew_ref[0], kr)
    hp1 = hp1_ref[0, 0]
    hp2 = hp2_ref[0, 0]
    hb1 = hb1_ref[0, 0]
    hb2 = hb2_ref[0, 0]
    dx = jax.nn.silu(jnp.dot(kr, hp1, preferred_element_type=jnp.float32) + hb1)
    h1 = dx + kr
    kh = jnp.sign(jnp.dot(h1, hp2, preferred_element_type=jnp.float32) + hb2 + h1)
    qr = qr_ref[0]
    qh = qh_ref[0]
    draft_ref[0] = _dot_t(qh, kh)
    score_ref[0] = _dot_t(qr, kr) * (1.0 / np.sqrt(DH))


def _select_kernel(draft_ref, score_ref, w_ref):
    draft = draft_ref[:, 0, :]
    col = jax.lax.broadcasted_iota(jnp.int32, (H, S), 1).astype(jnp.float32)
    comp = draft * S + (S - 1 - col)
    lo = jnp.full((H, 1), -float(2 ** 20), jnp.float32)
    hi = jnp.full((H, 1), float(2 ** 20), jnp.float32)
    for _ in range(22):
        mid = jnp.floor((lo + hi) * 0.5)
        cnt = jnp.sum((comp >= mid).astype(jnp.float32), axis=1, keepdims=True)
        ok = cnt >= NUM_REMAIN
        lo = jnp.where(ok, mid, lo)
        hi = jnp.where(ok, hi, mid)
    sel = comp >= lo
    sc = score_ref[:, 0, :]
    m = jnp.max(jnp.where(sel, sc, -jnp.inf), axis=1, keepdims=True)
    p = jnp.where(sel, jnp.exp(sc - m), 0.0)
    w_ref[:, 0, :] = p / jnp.sum(p, axis=1, keepdims=True)


def _ctxo_kernel(w_ref, v_ref, vnew_ref, wo_ref, hid_ref, h2_ref):
    h = pl.program_id(0)
    w = w_ref[0]
    vb = v_ref[0, 0]
    ctx = jax.lax.dot_general(w[:, :KV_LEN], vb, (((1,), (0,)), ((), ())),
                              preferred_element_type=jnp.float32)
    ctx = ctx + w[:, KV_LEN:] * vnew_ref[0]
    part = jnp.dot(ctx, wo_ref[...], preferred_element_type=jnp.float32)

    @pl.when(h == 0)
    def _():
        h2_ref[...] = hid_ref[...] + part

    @pl.when(h > 0)
    def _():
        h2_ref[...] += part


GKB = 128
DKB = 256


def _gate_kernel(h2f_ref, h2b_ref, ln2_ref, wg_ref, wu_ref, a_ref, b_ref):
    k = pl.program_id(0)
    h2 = h2f_ref[...]
    s = jax.lax.rsqrt(jnp.mean(h2 * h2) + _EPS)
    h3k = h2b_ref[...] * s * ln2_ref[...]
    ap = jnp.dot(h3k, wg_ref[...], preferred_element_type=jnp.float32)
    bp = jnp.dot(h3k, wu_ref[...], preferred_element_type=jnp.float32)

    @pl.when(k == 0)
    def _():
        a_ref[...] = ap
        b_ref[...] = bp

    @pl.when(k > 0)
    def _():
        a_ref[...] += ap
        b_ref[...] += bp


def _down_kernel(a_ref, b_ref, wd_ref, h2_ref, out_ref):
    k = pl.program_id(0)
    g = jax.nn.silu(a_ref[...]) * b_ref[...]
    part = jnp.dot(g, wd_ref[...], preferred_element_type=jnp.float32)

    @pl.when(k == 0)
    def _():
        out_ref[...] = h2_ref[...] + part

    @pl.when(k > 0)
    def _():
        out_ref[...] += part


def kernel(hidden_states, key_cache, val_cache, Wq, Wk, Wv, Wo, ln1_w, ln2_w,
           Wg, Wu, Wd, hp1, hb1, hp2, hb2):
    f32 = jnp.float32
    cos_np, sin_np = _rope_tables_np()
    cos = jnp.asarray(cos_np)
    sin = jnp.asarray(sin_np)
    cos_q = cos[S - 1:S]
    sin_q = sin[S - 1:S]

    hid = hidden_states.reshape(1, D)
    ln1 = ln1_w.reshape(1, D)
    ln2 = ln2_w.reshape(1, D)

    q_r, k_new, v_new, qh_all = pl.pallas_call(
        _qkv_kernel,
        grid=(H // QB,),
        in_specs=[
            pl.BlockSpec((1, D), lambda h: (0, 0)),
            pl.BlockSpec((1, D), lambda h: (0, 0)),
            pl.BlockSpec((D, QB * DH), lambda h: (0, h)),
            pl.BlockSpec((D, QB * DH), lambda h: (0, h)),
            pl.BlockSpec((D, QB * DH), lambda h: (0, h)),
            pl.BlockSpec((1, DH), lambda h: (0, 0)),
            pl.BlockSpec((1, DH), lambda h: (0, 0)),
            pl.BlockSpec((1, QB, DH, DH), lambda h: (0, h, 0, 0)),
            pl.BlockSpec((1, QB, 1, DH), lambda h: (0, h, 0, 0)),
            pl.BlockSpec((1, QB, DH, DH), lambda h: (0, h, 0, 0)),
            pl.BlockSpec((1, QB, 1, DH), lambda h: (0, h, 0, 0)),
        ],
        out_specs=[pl.BlockSpec((QB, 1, DH), lambda h: (h, 0, 0))] * 4,
        out_shape=[jax.ShapeDtypeStruct((H, 1, DH), f32)] * 4,
    )(hid, ln1, Wq, Wk, Wv, cos_q, sin_q, hp1, hb1, hp2, hb2)

    nkb = S // KB
    draft, score = pl.pallas_call(
        _key_kernel,
        grid=(H, nkb),
        in_specs=[
            pl.BlockSpec((1, 1, KB, DH), lambda h, b: (0, h, b, 0)),
            pl.BlockSpec((KB, DH), lambda h, b: (b, 0)),
            pl.BlockSpec((KB, DH), lambda h, b: (b, 0)),
            pl.BlockSpec((1, 1, DH, DH), lambda h, b: (0, h, 0, 0)),
            pl.BlockSpec((1, 1, 1, DH), lambda h, b: (0, h, 0, 0)),
            pl.BlockSpec((1, 1, DH, DH), lambda h, b: (0, h, 0, 0)),
            pl.BlockSpec((1, 1, 1, DH), lambda h, b: (0, h, 0, 0)),
            pl.BlockSpec((1, 1, DH), lambda h, b: (h, 0, 0)),
            pl.BlockSpec((1, 1, DH), lambda h, b: (h, 0, 0)),
            pl.BlockSpec((1, 1, DH), lambda h, b: (h, 0, 0)),
        ],
        out_specs=[pl.BlockSpec((1, 1, KB), lambda h, b: (h, 0, b))] * 2,
        out_shape=[jax.ShapeDtypeStruct((H, 1, S), f32)] * 2,
    )(key_cache, cos, sin, hp1, hb1, hp2, hb2, q_r, k_new, qh_all)

    w = pl.pallas_call(
        _select_kernel,
        in_specs=[
            pl.BlockSpec((H, 1, S), lambda: (0, 0, 0)),
            pl.BlockSpec((H, 1, S), lambda: (0, 0, 0)),
        ],
        out_specs=pl.BlockSpec((H, 1, S), lambda: (0, 0, 0)),
        out_shape=jax.ShapeDtypeStruct((H, 1, S), f32),
        grid=(),
    )(draft, score)

    h2 = pl.pallas_call(
        _ctxo_kernel,
        grid=(H,),
        in_specs=[
            pl.BlockSpec((1, 1, S), lambda h: (h, 0, 0)),
            pl.BlockSpec((1, 1, KV_LEN, DH), lambda h: (0, h, 0, 0)),
            pl.BlockSpec((1, 1, DH), lambda h: (h, 0, 0)),
            pl.BlockSpec((DH, D), lambda h: (h, 0)),
            pl.BlockSpec((1, D), lambda h: (0, 0)),
        ],
        out_specs=pl.BlockSpec((1, D), lambda h: (0, 0)),
        out_shape=jax.ShapeDtypeStruct((1, D), f32),
    )(w, val_cache, v_new, Wo, hid)

    a_acc, b_acc = pl.pallas_call(
        _gate_kernel,
        grid=(D // GKB,),
        in_specs=[
            pl.BlockSpec((1, D), lambda k: (0, 0)),
            pl.BlockSpec((1, GKB), lambda k: (0, k)),
            pl.BlockSpec((1, GKB), lambda k: (0, k)),
            pl.BlockSpec((GKB, DFF), lambda k: (k, 0)),
            pl.BlockSpec((GKB, DFF), lambda k: (k, 0)),
        ],
        out_specs=[pl.BlockSpec((1, DFF), lambda k: (0, 0))] * 2,
        out_shape=[jax.ShapeDtypeStruct((1, DFF), f32)] * 2,
    )(h2, h2, ln2, Wg, Wu)

    out = pl.pallas_call(
        _down_kernel,
        grid=(DFF // DKB,),
        in_specs=[
            pl.BlockSpec((1, DKB), lambda k: (0, k)),
            pl.BlockSpec((1, DKB), lambda k: (0, k)),
            pl.BlockSpec((DKB, D), lambda k: (k, 0)),
            pl.BlockSpec((1, D), lambda k: (0, 0)),
        ],
        out_specs=pl.BlockSpec((1, D), lambda k: (0, 0)),
        out_shape=jax.ShapeDtypeStruct((1, D), f32),
    )(a_acc, b_acc, Wd, h2)

    return out.reshape(1, 1, D)


# speedup vs baseline: 1.0873x; 1.0873x over previous
import numpy as np
import jax
import jax.numpy as jnp
from jax.experimental import pallas as pl

H = 32
DH = 128
D = 4096
DFF = 11008
KV_LEN = 4095
S = KV_LEN + 1
KB = 4096
NUM_REMAIN = max(min(S, 128), S - int(S * 0.9))
_EPS = 1e-5


def _rope_tables_np():
    inv_freq = 1.0 / (10000.0 ** (np.arange(0, DH, 2, dtype=np.float64) / DH))
    pos = np.arange(S, dtype=np.float64)
    freqs = np.outer(pos, inv_freq)
    emb = np.concatenate([freqs, freqs], axis=-1)
    sin = np.sin(emb)
    sin[:, : DH // 2] *= -1.0
    return np.cos(emb).astype(np.float32), sin.astype(np.float32)


def _roll(x):
    x1 = x[..., : DH // 2]
    x2 = x[..., DH // 2:]
    return jnp.concatenate([x2, x1], axis=-1)


def _dot_t(a, b):
    return jax.lax.dot_general(a, b, (((1,), (1,)), ((), ())),
                               preferred_element_type=jnp.float32)


QB = 2


def _qkv_kernel(hid_ref, ln1_ref, wq_ref, wk_ref, wv_ref, cos_ref, sin_ref,
                hp1_ref, hb1_ref, hp2_ref, hb2_ref,
                q_ref, k_ref, v_ref, qh_ref):
    h = hid_ref[...]
    s = jax.lax.rsqrt(jnp.mean(h * h) + _EPS)
    hn = h * s * ln1_ref[...]
    q = jnp.dot(hn, wq_ref[...], preferred_element_type=jnp.float32)
    k = jnp.dot(hn, wk_ref[...], preferred_element_type=jnp.float32)
    v = jnp.dot(hn, wv_ref[...], preferred_element_type=jnp.float32)
    c = cos_ref[...]
    sn = sin_ref[...]
    for i in range(QB):
        sl = slice(i * DH, (i + 1) * DH)
        qi, ki, vi = q[:, sl], k[:, sl], v[:, sl]
        qr = qi * c + _roll(qi) * sn
        q_ref[i] = qr
        k_ref[i] = ki * c + _roll(ki) * sn
        v_ref[i] = vi
        hp1 = hp1_ref[0, i]
        hp2 = hp2_ref[0, i]
        dq = jax.nn.silu(jnp.dot(qr, hp1, preferred_element_type=jnp.float32)
                         + hb1_ref[0, i])
        q1 = dq + qr
        qh_ref[i] = jnp.sign(jnp.dot(q1, hp2, preferred_element_type=jnp.float32)
                             + hb2_ref[0, i] + q1)


def _key_kernel(kc_ref, cos_ref, sin_ref, hp1_ref, hb1_ref, hp2_ref, hb2_ref,
                qr_ref, knew_ref, qh_ref, draft_ref, score_ref):
    b = pl.program_id(1)
    kb = kc_ref[0, 0]
    c = cos_ref[...]
    sn = sin_ref[...]
    kr = kb * c + _roll(kb) * sn
    rows = b * KB + jax.lax.broadcasted_iota(jnp.int32, (KB, 1), 0)
    kr = jnp.where(rows == S - 1, knew_ref[0], kr)
    hp1 = hp1_ref[0, 0]
    hp2 = hp2_ref[0, 0]
    hb1 = hb1_ref[0, 0]
    hb2 = hb2_ref[0, 0]
    dx = jax.nn.silu(jnp.dot(kr, hp1, preferred_element_type=jnp.float32) + hb1)
    h1 = dx + kr
    kh = jnp.sign(jnp.dot(h1, hp2, preferred_element_type=jnp.float32) + hb2 + h1)
    qr = qr_ref[0]
    qh = qh_ref[0]
    draft_ref[0] = _dot_t(qh, kh)
    score_ref[0] = _dot_t(qr, kr) * (1.0 / np.sqrt(DH))


def _select_kernel(draft_ref, score_ref, w_ref):
    draft = draft_ref[:, 0, :]
    col = jax.lax.broadcasted_iota(jnp.int32, (H, S), 1).astype(jnp.float32)
    comp = draft * S + (S - 1 - col)
    lo = jnp.full((H, 1), -float(2 ** 20), jnp.float32)
    hi = jnp.full((H, 1), float(2 ** 20), jnp.float32)
    for _ in range(22):
        mid = jnp.floor((lo + hi) * 0.5)
        cnt = jnp.sum((comp >= mid).astype(jnp.float32), axis=1, keepdims=True)
        ok = cnt >= NUM_REMAIN
        lo = jnp.where(ok, mid, lo)
        hi = jnp.where(ok, hi, mid)
    sel = comp >= lo
    sc = score_ref[:, 0, :]
    m = jnp.max(jnp.where(sel, sc, -jnp.inf), axis=1, keepdims=True)
    p = jnp.where(sel, jnp.exp(sc - m), 0.0)
    w_ref[:, 0, :] = p / jnp.sum(p, axis=1, keepdims=True)


def _ctxo_kernel(w_ref, v_ref, vnew_ref, wo_ref, hid_ref, h2_ref):
    h = pl.program_id(0)
    w = w_ref[0]
    vb = v_ref[0, 0]
    ctx = jax.lax.dot_general(w[:, :KV_LEN], vb, (((1,), (0,)), ((), ())),
                              preferred_element_type=jnp.float32)
    ctx = ctx + w[:, KV_LEN:] * vnew_ref[0]
    part = jnp.dot(ctx, wo_ref[...], preferred_element_type=jnp.float32)

    @pl.when(h == 0)
    def _():
        h2_ref[...] = hid_ref[...] + part

    @pl.when(h > 0)
    def _():
        h2_ref[...] += part


def _gate_kernel(h2_ref, ln2_ref, wg_ref, wu_ref, g_ref):
    h2 = h2_ref[...]
    s = jax.lax.rsqrt(jnp.mean(h2 * h2) + _EPS)
    h3 = h2 * s * ln2_ref[...]
    a = jnp.dot(h3, wg_ref[...], preferred_element_type=jnp.float32)
    bu = jnp.dot(h3, wu_ref[...], preferred_element_type=jnp.float32)
    g_ref[...] = jax.nn.silu(a) * bu


def _down_kernel(g_ref, wd_ref, h2_ref, out_ref):
    out_ref[...] = (jnp.dot(g_ref[...], wd_ref[...],
                            preferred_element_type=jnp.float32) + h2_ref[...])


def kernel(hidden_states, key_cache, val_cache, Wq, Wk, Wv, Wo, ln1_w, ln2_w,
           Wg, Wu, Wd, hp1, hb1, hp2, hb2):
    f32 = jnp.float32
    cos_np, sin_np = _rope_tables_np()
    cos = jnp.asarray(cos_np)
    sin = jnp.asarray(sin_np)
    cos_q = cos[S - 1:S]
    sin_q = sin[S - 1:S]

    hid = hidden_states.reshape(1, D)
    ln1 = ln1_w.reshape(1, D)
    ln2 = ln2_w.reshape(1, D)

    q_r, k_new, v_new, qh_all = pl.pallas_call(
        _qkv_kernel,
        grid=(H // QB,),
        in_specs=[
            pl.BlockSpec((1, D), lambda h: (0, 0)),
            pl.BlockSpec((1, D), lambda h: (0, 0)),
            pl.BlockSpec((D, QB * DH), lambda h: (0, h)),
            pl.BlockSpec((D, QB * DH), lambda h: (0, h)),
            pl.BlockSpec((D, QB * DH), lambda h: (0, h)),
            pl.BlockSpec((1, DH), lambda h: (0, 0)),
            pl.BlockSpec((1, DH), lambda h: (0, 0)),
            pl.BlockSpec((1, QB, DH, DH), lambda h: (0, h, 0, 0)),
            pl.BlockSpec((1, QB, 1, DH), lambda h: (0, h, 0, 0)),
            pl.BlockSpec((1, QB, DH, DH), lambda h: (0, h, 0, 0)),
            pl.BlockSpec((1, QB, 1, DH), lambda h: (0, h, 0, 0)),
        ],
        out_specs=[pl.BlockSpec((QB, 1, DH), lambda h: (h, 0, 0))] * 4,
        out_shape=[jax.ShapeDtypeStruct((H, 1, DH), f32)] * 4,
    )(hid, ln1, Wq, Wk, Wv, cos_q, sin_q, hp1, hb1, hp2, hb2)

    nkb = S // KB
    draft, score = pl.pallas_call(
        _key_kernel,
        grid=(H, nkb),
        in_specs=[
            pl.BlockSpec((1, 1, KB, DH), lambda h, b: (0, h, b, 0)),
            pl.BlockSpec((KB, DH), lambda h, b: (b, 0)),
            pl.BlockSpec((KB, DH), lambda h, b: (b, 0)),
            pl.BlockSpec((1, 1, DH, DH), lambda h, b: (0, h, 0, 0)),
            pl.BlockSpec((1, 1, 1, DH), lambda h, b: (0, h, 0, 0)),
            pl.BlockSpec((1, 1, DH, DH), lambda h, b: (0, h, 0, 0)),
            pl.BlockSpec((1, 1, 1, DH), lambda h, b: (0, h, 0, 0)),
            pl.BlockSpec((1, 1, DH), lambda h, b: (h, 0, 0)),
            pl.BlockSpec((1, 1, DH), lambda h, b: (h, 0, 0)),
            pl.BlockSpec((1, 1, DH), lambda h, b: (h, 0, 0)),
        ],
        out_specs=[pl.BlockSpec((1, 1, KB), lambda h, b: (h, 0, b))] * 2,
        out_shape=[jax.ShapeDtypeStruct((H, 1, S), f32)] * 2,
    )(key_cache, cos, sin, hp1, hb1, hp2, hb2, q_r, k_new, qh_all)

    w = pl.pallas_call(
        _select_kernel,
        in_specs=[
            pl.BlockSpec((H, 1, S), lambda: (0, 0, 0)),
            pl.BlockSpec((H, 1, S), lambda: (0, 0, 0)),
        ],
        out_specs=pl.BlockSpec((H, 1, S), lambda: (0, 0, 0)),
        out_shape=jax.ShapeDtypeStruct((H, 1, S), f32),
        grid=(),
    )(draft, score)

    h2 = pl.pallas_call(
        _ctxo_kernel,
        grid=(H,),
        in_specs=[
            pl.BlockSpec((1, 1, S), lambda h: (h, 0, 0)),
            pl.BlockSpec((1, 1, KV_LEN, DH), lambda h: (0, h, 0, 0)),
            pl.BlockSpec((1, 1, DH), lambda h: (h, 0, 0)),
            pl.BlockSpec((DH, D), lambda h: (h, 0)),
            pl.BlockSpec((1, D), lambda h: (0, 0)),
        ],
        out_specs=pl.BlockSpec((1, D), lambda h: (0, 0)),
        out_shape=jax.ShapeDtypeStruct((1, D), f32),
    )(w, val_cache, v_new, Wo, hid)

    GB = 256
    g = pl.pallas_call(
        _gate_kernel,
        grid=(DFF // GB,),
        in_specs=[
            pl.BlockSpec((1, D), lambda j: (0, 0)),
            pl.BlockSpec((1, D), lambda j: (0, 0)),
            pl.BlockSpec((D, GB), lambda j: (0, j)),
            pl.BlockSpec((D, GB), lambda j: (0, j)),
        ],
        out_specs=pl.BlockSpec((1, GB), lambda j: (0, j)),
        out_shape=jax.ShapeDtypeStruct((1, DFF), f32),
    )(h2, ln2, Wg, Wu)

    DB = 256
    out = pl.pallas_call(
        _down_kernel,
        grid=(D // DB,),
        in_specs=[
            pl.BlockSpec((1, DFF), lambda j: (0, 0)),
            pl.BlockSpec((DFF, DB), lambda j: (0, j)),
            pl.BlockSpec((1, DB), lambda j: (0, j)),
        ],
        out_specs=pl.BlockSpec((1, DB), lambda j: (0, j)),
        out_shape=jax.ShapeDtypeStruct((1, D), f32),
    )(g, Wd, h2)

    return out.reshape(1, 1, D)


# speedup vs baseline: 1.0959x; 1.0080x over previous
import numpy as np
import jax
import jax.numpy as jnp
from jax.experimental import pallas as pl
from jax.experimental.pallas import tpu as pltpu

H = 32
DH = 128
D = 4096
DFF = 11008
KV_LEN = 4095
S = KV_LEN + 1
KB = 4096
NUM_REMAIN = max(min(S, 128), S - int(S * 0.9))
_EPS = 1e-5


def _rope_tables_np():
    inv_freq = 1.0 / (10000.0 ** (np.arange(0, DH, 2, dtype=np.float64) / DH))
    pos = np.arange(S, dtype=np.float64)
    freqs = np.outer(pos, inv_freq)
    emb = np.concatenate([freqs, freqs], axis=-1)
    sin = np.sin(emb)
    sin[:, : DH // 2] *= -1.0
    return np.cos(emb).astype(np.float32), sin.astype(np.float32)


def _roll(x):
    x1 = x[..., : DH // 2]
    x2 = x[..., DH // 2:]
    return jnp.concatenate([x2, x1], axis=-1)


def _dot_t(a, b):
    return jax.lax.dot_general(a, b, (((1,), (1,)), ((), ())),
                               preferred_element_type=jnp.float32)


QB = 2


def _qkv_kernel(hid_ref, ln1_ref, wq_ref, wk_ref, wv_ref, cos_ref, sin_ref,
                hp1_ref, hb1_ref, hp2_ref, hb2_ref,
                q_ref, k_ref, v_ref, qh_ref):
    h = hid_ref[...]
    s = jax.lax.rsqrt(jnp.mean(h * h) + _EPS)
    hn = h * s * ln1_ref[...]
    q = jnp.dot(hn, wq_ref[...], preferred_element_type=jnp.float32)
    k = jnp.dot(hn, wk_ref[...], preferred_element_type=jnp.float32)
    v = jnp.dot(hn, wv_ref[...], preferred_element_type=jnp.float32)
    c = cos_ref[...]
    sn = sin_ref[...]
    for i in range(QB):
        sl = slice(i * DH, (i + 1) * DH)
        qi, ki, vi = q[:, sl], k[:, sl], v[:, sl]
        qr = qi * c + _roll(qi) * sn
        q_ref[i] = qr
        k_ref[i] = ki * c + _roll(ki) * sn
        v_ref[i] = vi
        hp1 = hp1_ref[0, i]
        hp2 = hp2_ref[0, i]
        dq = jax.nn.silu(jnp.dot(qr, hp1, preferred_element_type=jnp.float32)
                         + hb1_ref[0, i])
        q1 = dq + qr
        qh_ref[i] = jnp.sign(jnp.dot(q1, hp2, preferred_element_type=jnp.float32)
                             + hb2_ref[0, i] + q1)


def _key_kernel(kc_ref, cos_ref, sin_ref, hp1_ref, hb1_ref, hp2_ref, hb2_ref,
                qr_ref, knew_ref, qh_ref, draft_ref, score_ref):
    b = pl.program_id(1)
    kb = kc_ref[0, 0]
    c = cos_ref[...]
    sn = sin_ref[...]
    kr = kb * c + _roll(kb) * sn
    rows = b * KB + jax.lax.broadcasted_iota(jnp.int32, (KB, 1), 0)
    kr = jnp.where(rows == S - 1, knew_ref[0], kr)
    hp1 = hp1_ref[0, 0]
    hp2 = hp2_ref[0, 0]
    hb1 = hb1_ref[0, 0]
    hb2 = hb2_ref[0, 0]
    dx = jax.nn.silu(jnp.dot(kr, hp1, preferred_element_type=jnp.float32) + hb1)
    h1 = dx + kr
    kh = jnp.sign(jnp.dot(h1, hp2, preferred_element_type=jnp.float32) + hb2 + h1)
    qr = qr_ref[0]
    qh = qh_ref[0]
    draft_ref[0] = _dot_t(qh, kh)
    score_ref[0] = _dot_t(qr, kr) * (1.0 / np.sqrt(DH))


def _select(draft, score):
    col = jax.lax.broadcasted_iota(jnp.int32, (H, S), 1).astype(jnp.float32)
    comp = draft * S + (S - 1 - col)
    lo = jnp.full((H, 1), -float(2 ** 20), jnp.float32)
    hi = jnp.full((H, 1), float(2 ** 20), jnp.float32)
    for _ in range(22):
        mid = jnp.floor((lo + hi) * 0.5)
        cnt = jnp.sum((comp >= mid).astype(jnp.float32), axis=1, keepdims=True)
        ok = cnt >= NUM_REMAIN
        lo = jnp.where(ok, mid, lo)
        hi = jnp.where(ok, hi, mid)
    sel = comp >= lo
    m = jnp.max(jnp.where(sel, score, -jnp.inf), axis=1, keepdims=True)
    p = jnp.where(sel, jnp.exp(score - m), 0.0)
    return p / jnp.sum(p, axis=1, keepdims=True)


def _ctxo_kernel(draft_ref, score_ref, v_ref, vnew_ref, wo_ref, hid_ref,
                 h2_ref, w_scr):
    h = pl.program_id(0)

    @pl.when(h == 0)
    def _():
        w_scr[...] = _select(draft_ref[:, 0, :], score_ref[:, 0, :])

    w = w_scr[pl.ds(h, 1), :]
    vb = v_ref[0, 0]
    ctx = jax.lax.dot_general(w[:, :KV_LEN], vb, (((1,), (0,)), ((), ())),
                              preferred_element_type=jnp.float32)
    ctx = ctx + w[:, KV_LEN:] * vnew_ref[0]
    part = jnp.dot(ctx, wo_ref[...], preferred_element_type=jnp.float32)

    @pl.when(h == 0)
    def _():
        h2_ref[...] = hid_ref[...] + part

    @pl.when(h > 0)
    def _():
        h2_ref[...] += part


def _gate_kernel(h2_ref, ln2_ref, wg_ref, wu_ref, g_ref):
    h2 = h2_ref[...]
    s = jax.lax.rsqrt(jnp.mean(h2 * h2) + _EPS)
    h3 = h2 * s * ln2_ref[...]
    a = jnp.dot(h3, wg_ref[...], preferred_element_type=jnp.float32)
    bu = jnp.dot(h3, wu_ref[...], preferred_element_type=jnp.float32)
    g_ref[...] = jax.nn.silu(a) * bu


def _down_kernel(g_ref, wd_ref, h2_ref, out_ref):
    out_ref[...] = (jnp.dot(g_ref[...], wd_ref[...],
                            preferred_element_type=jnp.float32) + h2_ref[...])


def kernel(hidden_states, key_cache, val_cache, Wq, Wk, Wv, Wo, ln1_w, ln2_w,
           Wg, Wu, Wd, hp1, hb1, hp2, hb2):
    f32 = jnp.float32
    cos_np, sin_np = _rope_tables_np()
    cos = jnp.asarray(cos_np)
    sin = jnp.asarray(sin_np)
    cos_q = cos[S - 1:S]
    sin_q = sin[S - 1:S]

    hid = hidden_states.reshape(1, D)
    ln1 = ln1_w.reshape(1, D)
    ln2 = ln2_w.reshape(1, D)

    q_r, k_new, v_new, qh_all = pl.pallas_call(
        _qkv_kernel,
        grid=(H // QB,),
        in_specs=[
            pl.BlockSpec((1, D), lambda h: (0, 0)),
            pl.BlockSpec((1, D), lambda h: (0, 0)),
            pl.BlockSpec((D, QB * DH), lambda h: (0, h)),
            pl.BlockSpec((D, QB * DH), lambda h: (0, h)),
            pl.BlockSpec((D, QB * DH), lambda h: (0, h)),
            pl.BlockSpec((1, DH), lambda h: (0, 0)),
            pl.BlockSpec((1, DH), lambda h: (0, 0)),
            pl.BlockSpec((1, QB, DH, DH), lambda h: (0, h, 0, 0)),
            pl.BlockSpec((1, QB, 1, DH), lambda h: (0, h, 0, 0)),
            pl.BlockSpec((1, QB, DH, DH), lambda h: (0, h, 0, 0)),
            pl.BlockSpec((1, QB, 1, DH), lambda h: (0, h, 0, 0)),
        ],
        out_specs=[pl.BlockSpec((QB, 1, DH), lambda h: (h, 0, 0))] * 4,
        out_shape=[jax.ShapeDtypeStruct((H, 1, DH), f32)] * 4,
    )(hid, ln1, Wq, Wk, Wv, cos_q, sin_q, hp1, hb1, hp2, hb2)

    nkb = S // KB
    draft, score = pl.pallas_call(
        _key_kernel,
        grid=(H, nkb),
        in_specs=[
            pl.BlockSpec((1, 1, KB, DH), lambda h, b: (0, h, b, 0)),
            pl.BlockSpec((KB, DH), lambda h, b: (b, 0)),
            pl.BlockSpec((KB, DH), lambda h, b: (b, 0)),
            pl.BlockSpec((1, 1, DH, DH), lambda h, b: (0, h, 0, 0)),
            pl.BlockSpec((1, 1, 1, DH), lambda h, b: (0, h, 0, 0)),
            pl.BlockSpec((1, 1, DH, DH), lambda h, b: (0, h, 0, 0)),
            pl.BlockSpec((1, 1, 1, DH), lambda h, b: (0, h, 0, 0)),
            pl.BlockSpec((1, 1, DH), lambda h, b: (h, 0, 0)),
            pl.BlockSpec((1, 1, DH), lambda h, b: (h, 0, 0)),
            pl.BlockSpec((1, 1, DH), lambda h, b: (h, 0, 0)),
        ],
        out_specs=[pl.BlockSpec((1, 1, KB), lambda h, b: (h, 0, b))] * 2,
        out_shape=[jax.ShapeDtypeStruct((H, 1, S), f32)] * 2,
    )(key_cache, cos, sin, hp1, hb1, hp2, hb2, q_r, k_new, qh_all)

    h2 = pl.pallas_call(
        _ctxo_kernel,
        grid=(H,),
        in_specs=[
            pl.BlockSpec((H, 1, S), lambda h: (0, 0, 0)),
            pl.BlockSpec((H, 1, S), lambda h: (0, 0, 0)),
            pl.BlockSpec((1, 1, KV_LEN, DH), lambda h: (0, h, 0, 0)),
            pl.BlockSpec((1, 1, DH), lambda h: (h, 0, 0)),
            pl.BlockSpec((DH, D), lambda h: (h, 0)),
            pl.BlockSpec((1, D), lambda h: (0, 0)),
        ],
        out_specs=pl.BlockSpec((1, D), lambda h: (0, 0)),
        out_shape=jax.ShapeDtypeStruct((1, D), f32),
        scratch_shapes=[pltpu.VMEM((H, S), f32)],
    )(draft, score, val_cache, v_new, Wo, hid)

    GB = 256
    g = pl.pallas_call(
        _gate_kernel,
        grid=(DFF // GB,),
        in_specs=[
            pl.BlockSpec((1, D), lambda j: (0, 0)),
            pl.BlockSpec((1, D), lambda j: (0, 0)),
            pl.BlockSpec((D, GB), lambda j: (0, j)),
            pl.BlockSpec((D, GB), lambda j: (0, j)),
        ],
        out_specs=pl.BlockSpec((1, GB), lambda j: (0, j)),
        out_shape=jax.ShapeDtypeStruct((1, DFF), f32),
    )(h2, ln2, Wg, Wu)

    DB = 256
    out = pl.pallas_call(
        _down_kernel,
        grid=(D // DB,),
        in_specs=[
            pl.BlockSpec((1, DFF), lambda j: (0, 0)),
            pl.BlockSpec((DFF, DB), lambda j: (0, j)),
            pl.BlockSpec((1, DB), lambda j: (0, j)),
        ],
        out_specs=pl.BlockSpec((1, DB), lambda j: (0, j)),
        out_shape=jax.ShapeDtypeStruct((1, D), f32),
    )(g, Wd, h2)

    return out.reshape(1, 1, D)
